# baseline (speedup 1.0000x reference)
import functools

import jax
import jax.numpy as jnp
from jax import lax
from jax.experimental import pallas as pl
from jax.experimental.pallas import tpu as pltpu
from jax.experimental.pallas import tpu_sc as plsc

_B = 4096
_F = 26
_D = 64
_FD = _F * _D

_NC = 2
_NS = 16
_NW = _NC * _NS
_BPW = _B // _NW
_RPW = _BPW * _F
_CHUNK = 128
_NCHUNK = _RPW // _CHUNK


@functools.partial(jax.jit, static_argnums=())
def _sc_gather(x_r, xt_r, emb_w, fc_flat):
    mesh = plsc.VectorSubcoreMesh(core_axis_name="c", subcore_axis_name="s")

    @functools.partial(
        pl.kernel,
        mesh=mesh,
        compiler_params=pltpu.CompilerParams(use_tc_tiling_on_sc=False),
        out_type=[
            jax.ShapeDtypeStruct((_B * _F, _D), jnp.float32),
            jax.ShapeDtypeStruct((_B,), jnp.float32),
        ],
        scratch_types=[
            pltpu.VMEM((_NCHUNK, _CHUNK), jnp.int32),
            pltpu.VMEM((_NCHUNK, _CHUNK), jnp.int32),
            pltpu.VMEM((2, _CHUNK, _D), jnp.float32),
            pltpu.VMEM((_F, _BPW), jnp.float32),
            pltpu.VMEM((_BPW,), jnp.float32),
            pltpu.SemaphoreType.DMA,
            pltpu.SemaphoreType.DMA,
            pltpu.SemaphoreType.DMA,
        ],
    )
    def body(x_hbm, xt_hbm, emb_hbm, fc_hbm, rows_out, fc_out,
             idx_v, idxt_v, rowbuf, fcbuf, accv, gsem, fsem, wsem):
        wid = lax.axis_index("s") * _NC + lax.axis_index("c")
        pltpu.sync_copy(x_hbm.at[wid], idx_v)
        pltpu.sync_copy(xt_hbm.at[wid], idxt_v)
        row0 = wid * _RPW

        def chunk_step(j, carry):
            hr = pltpu.async_copy(emb_hbm.at[idx_v.at[j]], rowbuf.at[0], gsem)
            hf = pltpu.async_copy(fc_hbm.at[idxt_v.at[j]], fcbuf.at[j], fsem)
            hr.wait()
            pltpu.sync_copy(rowbuf.at[0],
                            rows_out.at[pl.ds(row0 + j * _CHUNK, _CHUNK)])
            hf.wait()
            return carry

        lax.fori_loop(0, _NCHUNK, chunk_step, 0)

        for g in range(_BPW // 16):
            acc = jnp.zeros((16,), jnp.float32)
            for f in range(_F):
                acc = acc + fcbuf[f, pl.ds(g * 16, 16)]
            accv[pl.ds(g * 16, 16)] = acc
        pltpu.sync_copy(accv, fc_out.at[pl.ds(wid * _BPW, _BPW)])

    return body(x_r, xt_r, emb_w, fc_flat)


def _tc_fused(embed, fcsum, bias, W0, b0, W1, b1, W2, b2, W3, b3):
    bb = 512
    grid = (_B // bb,)

    def body(e_ref, fcsum_ref, bias_ref, b3s_ref,
             w0_ref, b0_ref, w1_ref, b1_ref, w2_ref, b2_ref, w3_ref, o_ref):
        e = e_ref[...]
        s_pair = jnp.zeros((bb, 2 * _D), jnp.float32)
        for k in range(_F // 2):
            s_pair = s_pair + e[:, k * 128:(k + 1) * 128]
        s = s_pair[:, :_D] + s_pair[:, _D:]
        sq_tot = jnp.sum(e * e, axis=1)
        fm2 = 0.5 * (jnp.sum(s * s, axis=1) - sq_tot)

        h = e
        for w_ref, b_ref in ((w0_ref, b0_ref), (w1_ref, b1_ref),
                             (w2_ref, b2_ref)):
            h = lax.dot_general(h, w_ref[...], (((1,), (1,)), ((), ())),
                                preferred_element_type=jnp.float32)
            h = jnp.maximum(h + b_ref[...][None, :], 0.0)
        mlp = jnp.sum(h * w3_ref[...], axis=1)

        z = fcsum_ref[...] + bias_ref[0] + b3s_ref[0] + fm2 + mlp
        o_ref[...] = 1.0 / (1.0 + jnp.exp(-z))

    return pl.pallas_call(
        body,
        grid=grid,
        in_specs=[
            pl.BlockSpec((bb, _FD), lambda i: (i, 0)),
            pl.BlockSpec((bb,), lambda i: (i,)),
            pl.BlockSpec(memory_space=pltpu.SMEM),
            pl.BlockSpec(memory_space=pltpu.SMEM),
            pl.BlockSpec((1024, _FD), lambda i: (0, 0)),
            pl.BlockSpec((1024,), lambda i: (0,)),
            pl.BlockSpec((512, 1024), lambda i: (0, 0)),
            pl.BlockSpec((512,), lambda i: (0,)),
            pl.BlockSpec((256, 512), lambda i: (0, 0)),
            pl.BlockSpec((256,), lambda i: (0,)),
            pl.BlockSpec((1, 256), lambda i: (0, 0)),
        ],
        out_specs=pl.BlockSpec((bb,), lambda i: (i,)),
        out_shape=jax.ShapeDtypeStruct((_B,), jnp.float32),
    )(embed, fcsum, bias, b3, W0, b0, W1, b1, W2, b2, W3)


def kernel(x, bias, fc_w, emb_w, W0, b0, W1, b1, W2, b2, W3, b3):
    xi = x.astype(jnp.int32)
    x_r = xi.reshape(_NW, _NCHUNK, _CHUNK)
    xt_r = xi.reshape(_NW, _BPW, _F).transpose(0, 2, 1)
    fc_flat = fc_w.reshape(-1)
    rows, fcsum = _sc_gather(x_r, xt_r, emb_w, fc_flat)
    embed = rows.reshape(_B, _FD)
    return _tc_fused(embed, fcsum, bias, W0, b0, W1, b1, W2, b2, W3, b3)
